# Initial kernel scaffold; baseline (speedup 1.0000x reference)
import jax
import jax.numpy as jnp
from jax.experimental import pallas as pl


def kernel(x, edge_index, W1, b1, W2, b2):
    raise NotImplementedError("write your pallas kernel here")


# speedup vs baseline: 16.6646x; 16.6646x over previous
import functools

import jax
import jax.numpy as jnp
from jax import lax
from jax.experimental import pallas as pl
from jax.experimental.pallas import tpu as pltpu
from jax.experimental.pallas import tpu_sc as plsc

N = 10000
E = 160000
F_IN = 768
F_HID = 16

NC = 2
NS = 16
NW = NC * NS
CHUNK = 128
CHUNKS_PW = 40
E_PW = CHUNK * CHUNKS_PW
EPAD = NW * E_PW
NPAD = 10240
ROWS_PT = NPAD // NS
DUMMY = 10200

MM_BLK = 640


def _mm1_body(x_ref, w_ref, o_ref):
    o_ref[...] = jnp.dot(x_ref[...], w_ref[...],
                         preferred_element_type=jnp.float32)


def _combine_body(p_ref, b_ref, o_ref):
    o_ref[...] = jnp.maximum(p_ref[0] + p_ref[1] + b_ref[...], 0.0)


def _final_body(q_ref, w_ref, b_ref, o_ref):
    agg = q_ref[0] + q_ref[1]
    o_ref[...] = jnp.dot(agg, w_ref[...],
                         preferred_element_type=jnp.float32) + b_ref[...]


def _make_segsum():
    mesh = plsc.VectorSubcoreMesh(core_axis_name="c", subcore_axis_name="s")

    @functools.partial(
        pl.kernel,
        mesh=mesh,
        out_type=jax.ShapeDtypeStruct((NC, NPAD, F_HID), jnp.float32),
        compiler_params=pltpu.CompilerParams(use_tc_tiling_on_sc=False),
        scratch_types=[
            pltpu.VMEM((CHUNK,), jnp.int32),
            pltpu.VMEM((CHUNK,), jnp.int32),
            pltpu.VMEM((CHUNK, F_HID), jnp.float32),
            pltpu.VMEM((ROWS_PT, F_HID), jnp.float32),
            pltpu.VMEM_SHARED((NPAD, F_HID), jnp.float32),
            pltpu.VMEM_SHARED((NPAD, F_HID), jnp.float32),
            pltpu.SemaphoreType.DMA,
        ],
    )
    def segsum(table_hbm, src_hbm, dst_hbm, out_hbm,
               idx_s, idx_d, rows, zbuf, acc, tab, sem):
        c = lax.axis_index("c")
        s = lax.axis_index("s")
        w = c * NS + s
        sl = pl.ds(s * ROWS_PT, ROWS_PT)

        pltpu.sync_copy(table_hbm.at[sl], tab.at[sl])

        zero = jnp.zeros((F_HID,), jnp.float32)

        def zfill(i, carry):
            zbuf[i, :] = zero
            return carry

        lax.fori_loop(0, ROWS_PT, zfill, 0)
        pltpu.sync_copy(zbuf, acc.at[sl])
        plsc.subcore_barrier()

        def edge_chunk(j, carry):
            pltpu.sync_copy(src_hbm.at[w, j], idx_s)
            pltpu.sync_copy(dst_hbm.at[w, j], idx_d)
            pltpu.async_copy(tab.at[idx_s], rows, sem).wait()
            pltpu.sync_copy(rows, acc.at[idx_d], add=True)
            return carry

        lax.fori_loop(0, CHUNKS_PW, edge_chunk, 0)
        plsc.subcore_barrier()

        pltpu.sync_copy(acc.at[sl], out_hbm.at[c, sl])

    return segsum


_segsum = _make_segsum()


def kernel(x, edge_index, W1, b1, W2, b2):
    src = edge_index[0].astype(jnp.int32)
    dst = edge_index[1].astype(jnp.int32)
    pad = EPAD - E
    srcp = jnp.concatenate([src, jnp.zeros((pad,), jnp.int32)])
    dstp = jnp.concatenate([dst, jnp.full((pad,), DUMMY, jnp.int32)])
    src3 = srcp.reshape(NW, CHUNKS_PW, CHUNK)
    dst3 = dstp.reshape(NW, CHUNKS_PW, CHUNK)
    b1r = b1.reshape(1, F_HID)
    b2r = b2.reshape(1, -1)

    y = pl.pallas_call(
        _mm1_body,
        grid=(NPAD // MM_BLK,),
        in_specs=[
            pl.BlockSpec((MM_BLK, F_IN), lambda i: (i, 0)),
            pl.BlockSpec((F_IN, F_HID), lambda i: (0, 0)),
        ],
        out_specs=pl.BlockSpec((MM_BLK, F_HID), lambda i: (i, 0)),
        out_shape=jax.ShapeDtypeStruct((NPAD, F_HID), jnp.float32),
    )(x, W1)

    p = _segsum(y, src3, dst3)

    h1 = pl.pallas_call(
        _combine_body,
        out_shape=jax.ShapeDtypeStruct((NPAD, F_HID), jnp.float32),
    )(p, b1r)

    q = _segsum(h1, src3, dst3)

    out = pl.pallas_call(
        _final_body,
        out_shape=jax.ShapeDtypeStruct((NPAD, b2.shape[0]), jnp.float32),
    )(q, W2, b2r)
    return out[:N]


# speedup vs baseline: 29.4567x; 1.7676x over previous
import functools

import jax
import jax.numpy as jnp
from jax import lax
from jax.experimental import pallas as pl
from jax.experimental.pallas import tpu as pltpu
from jax.experimental.pallas import tpu_sc as plsc

N = 10000
E = 160000
F_IN = 768
F_HID = 16

NC = 2
NS = 16
NW = NC * NS
CHUNK = 128
CHUNKS_PW = 40
E_PW = CHUNK * CHUNKS_PW
EPAD = NW * E_PW
NPAD = 10240
ROWS_PT = NPAD // NS
DUMMY = 10200

MM_BLK = 640


def _mm1_body(x_ref, w_ref, o_ref):
    o_ref[...] = jnp.dot(x_ref[...], w_ref[...],
                         preferred_element_type=jnp.float32)


def _final_body(q_ref, w_ref, b_ref, o_ref):
    agg = q_ref[0] + q_ref[1]
    o_ref[...] = jnp.dot(agg, w_ref[...],
                         preferred_element_type=jnp.float32) + b_ref[...]


def _make_segsum(fuse_combine):
    mesh = plsc.VectorSubcoreMesh(core_axis_name="c", subcore_axis_name="s")

    scratch = [
        pltpu.VMEM((CHUNKS_PW, CHUNK), jnp.int32),
        pltpu.VMEM((CHUNKS_PW, CHUNK), jnp.int32),
        pltpu.VMEM((CHUNK, F_HID), jnp.float32),
        pltpu.VMEM((CHUNK, F_HID), jnp.float32),
        pltpu.VMEM((ROWS_PT, F_HID), jnp.float32),
        pltpu.VMEM_SHARED((NPAD, F_HID), jnp.float32),
        pltpu.VMEM_SHARED((NPAD, F_HID), jnp.float32),
        pltpu.SemaphoreType.DMA,
        pltpu.SemaphoreType.DMA,
    ]
    if fuse_combine:
        scratch.insert(5, pltpu.VMEM((ROWS_PT, F_HID), jnp.float32))
        scratch.insert(6, pltpu.VMEM((F_HID,), jnp.float32))

    @functools.partial(
        pl.kernel,
        mesh=mesh,
        out_type=jax.ShapeDtypeStruct((NC, NPAD, F_HID), jnp.float32),
        compiler_params=pltpu.CompilerParams(use_tc_tiling_on_sc=False),
        scratch_types=scratch,
    )
    def segsum(*refs):
        if fuse_combine:
            (p_hbm, b_hbm, src_hbm, dst_hbm, out_hbm,
             idx_s, idx_d, rows0, rows1, zbuf, pbuf, bbuf, acc, tab,
             sem0, sem1) = refs
        else:
            (table_hbm, src_hbm, dst_hbm, out_hbm,
             idx_s, idx_d, rows0, rows1, zbuf, acc, tab,
             sem0, sem1) = refs
        c = lax.axis_index("c")
        s = lax.axis_index("s")
        w = c * NS + s
        sl = pl.ds(s * ROWS_PT, ROWS_PT)

        pltpu.sync_copy(src_hbm.at[w], idx_s)
        pltpu.sync_copy(dst_hbm.at[w], idx_d)

        if fuse_combine:
            pltpu.sync_copy(p_hbm.at[0, sl], zbuf)
            pltpu.sync_copy(p_hbm.at[1, sl], pbuf)
            pltpu.sync_copy(b_hbm, bbuf)
            bias = bbuf[...]

            def combine(i, carry):
                zbuf[i, :] = jnp.maximum(zbuf[i, :] + pbuf[i, :] + bias, 0.0)
                return carry

            lax.fori_loop(0, ROWS_PT, combine, 0)
            pltpu.sync_copy(zbuf, tab.at[sl])
        else:
            pltpu.sync_copy(table_hbm.at[sl], tab.at[sl])

        zero = jnp.zeros((F_HID,), jnp.float32)

        def zfill(i, carry):
            zbuf[i, :] = zero
            return carry

        lax.fori_loop(0, ROWS_PT, zfill, 0)
        pltpu.sync_copy(zbuf, acc.at[sl])
        plsc.subcore_barrier()

        pltpu.async_copy(tab.at[idx_s.at[0]], rows0, sem0)

        def edge_pair(g, carry):
            j0 = 2 * g
            j1 = j0 + 1
            cp1 = pltpu.async_copy(tab.at[idx_s.at[j1]], rows1, sem1)
            pltpu.make_async_copy(tab.at[idx_s.at[j0]], rows0, sem0).wait()
            pltpu.sync_copy(rows0, acc.at[idx_d.at[j0]], add=True)

            @pl.when(g < CHUNKS_PW // 2 - 1)
            def _():
                pltpu.async_copy(tab.at[idx_s.at[j0 + 2]], rows0, sem0)

            cp1.wait()
            pltpu.sync_copy(rows1, acc.at[idx_d.at[j1]], add=True)
            return carry

        lax.fori_loop(0, CHUNKS_PW // 2, edge_pair, 0)
        plsc.subcore_barrier()

        pltpu.sync_copy(acc.at[sl], out_hbm.at[c, sl])

    return segsum


_segsum1 = _make_segsum(fuse_combine=False)
_segsum2 = _make_segsum(fuse_combine=True)


def kernel(x, edge_index, W1, b1, W2, b2):
    src = edge_index[0].astype(jnp.int32)
    dst = edge_index[1].astype(jnp.int32)
    pad = EPAD - E
    srcp = jnp.concatenate([src, jnp.zeros((pad,), jnp.int32)])
    dstp = jnp.concatenate([dst, jnp.full((pad,), DUMMY, jnp.int32)])
    src3 = srcp.reshape(NW, CHUNKS_PW, CHUNK)
    dst3 = dstp.reshape(NW, CHUNKS_PW, CHUNK)
    b2r = b2.reshape(1, -1)

    y = pl.pallas_call(
        _mm1_body,
        grid=(NPAD // MM_BLK,),
        in_specs=[
            pl.BlockSpec((MM_BLK, F_IN), lambda i: (i, 0)),
            pl.BlockSpec((F_IN, F_HID), lambda i: (0, 0)),
        ],
        out_specs=pl.BlockSpec((MM_BLK, F_HID), lambda i: (i, 0)),
        out_shape=jax.ShapeDtypeStruct((NPAD, F_HID), jnp.float32),
    )(x, W1)

    p = _segsum1(y, src3, dst3)

    q = _segsum2(p, b1, src3, dst3)

    out = pl.pallas_call(
        _final_body,
        out_shape=jax.ShapeDtypeStruct((NPAD, b2.shape[0]), jnp.float32),
    )(q, W2, b2r)
    return out[:N]


# speedup vs baseline: 31.6444x; 1.0743x over previous
import functools

import jax
import jax.numpy as jnp
from jax import lax
from jax.experimental import pallas as pl
from jax.experimental.pallas import tpu as pltpu
from jax.experimental.pallas import tpu_sc as plsc

N = 10000
E = 160000
F_IN = 768
F_HID = 16

NC = 2
NS = 16
NW = NC * NS
CHUNK = 128
CHUNKS_PW = 40
E_PW = CHUNK * CHUNKS_PW
EPAD = NW * E_PW
NPAD = 10240
ROWS_PT = NPAD // NS
DUMMY = 10200

MM_BLK = 640


def _mm1_body(x_ref, w_ref, o_ref):
    o_ref[...] = jnp.dot(x_ref[...], w_ref[...],
                         preferred_element_type=jnp.float32)


def _final_body(q_ref, w_ref, b_ref, o_ref):
    agg = q_ref[0] + q_ref[1]
    o_ref[...] = jnp.dot(agg, w_ref[...],
                         preferred_element_type=jnp.float32) + b_ref[...]


def _make_segsum(fuse_combine):
    mesh = plsc.VectorSubcoreMesh(core_axis_name="c", subcore_axis_name="s")

    scratch = [
        pltpu.VMEM((CHUNKS_PW, CHUNK), jnp.int32),
        pltpu.VMEM((CHUNKS_PW, CHUNK), jnp.int32),
        [pltpu.VMEM((CHUNK, F_HID), jnp.float32)] * 4,
        pltpu.VMEM((ROWS_PT, F_HID), jnp.float32),
        pltpu.VMEM_SHARED((NPAD, F_HID), jnp.float32),
        pltpu.VMEM_SHARED((NPAD, F_HID), jnp.float32),
        [pltpu.SemaphoreType.DMA] * 4,
        [pltpu.SemaphoreType.DMA] * 4,
    ]
    if fuse_combine:
        scratch.insert(4, pltpu.VMEM((ROWS_PT, F_HID), jnp.float32))
        scratch.insert(5, pltpu.VMEM((ROWS_PT, F_HID), jnp.float32))
        scratch.insert(6, pltpu.VMEM((F_HID,), jnp.float32))

    @functools.partial(
        pl.kernel,
        mesh=mesh,
        out_type=jax.ShapeDtypeStruct((NC, NPAD, F_HID), jnp.float32),
        compiler_params=pltpu.CompilerParams(use_tc_tiling_on_sc=False),
        scratch_types=scratch,
    )
    def segsum(*refs):
        if fuse_combine:
            (p_hbm, b_hbm, src_hbm, dst_hbm, out_hbm,
             idx_s, idx_d, rows, zbuf, pb0, pb1, bbuf, acc, tab,
             gsem, ssem) = refs
        else:
            (table_hbm, src_hbm, dst_hbm, out_hbm,
             idx_s, idx_d, rows, zbuf, acc, tab,
             gsem, ssem) = refs
        c = lax.axis_index("c")
        s = lax.axis_index("s")
        w = c * NS + s
        sl = pl.ds(s * ROWS_PT, ROWS_PT)

        cp_is = pltpu.async_copy(src_hbm.at[w], idx_s, gsem[0])
        cp_id = pltpu.async_copy(dst_hbm.at[w], idx_d, gsem[1])
        if fuse_combine:
            cp_p0 = pltpu.async_copy(p_hbm.at[0, sl], pb0, gsem[2])
            cp_p1 = pltpu.async_copy(p_hbm.at[1, sl], pb1, gsem[3])
            cp_b = pltpu.async_copy(b_hbm, bbuf, ssem[0])
        else:
            cp_tab = pltpu.async_copy(table_hbm.at[sl], tab.at[sl], gsem[2])

        zero = jnp.zeros((F_HID,), jnp.float32)

        def zfill(i, carry):
            zbuf[i, :] = zero
            return carry

        lax.fori_loop(0, ROWS_PT, zfill, 0)

        if fuse_combine:
            cp_p0.wait()
            cp_p1.wait()
            cp_b.wait()
            bias = bbuf[...]

            def combine(i, carry):
                pb0[i, :] = jnp.maximum(pb0[i, :] + pb1[i, :] + bias, 0.0)
                return carry

            lax.fori_loop(0, ROWS_PT, combine, 0)
            pltpu.sync_copy(pb0, tab.at[sl])
        else:
            cp_tab.wait()
        cp_is.wait()
        cp_id.wait()
        pltpu.sync_copy(zbuf, acc.at[sl])
        plsc.subcore_barrier()

        def gather(j, b):
            return pltpu.async_copy(tab.at[idx_s.at[j]], rows[b], gsem[b])

        def scatter(j, b):
            return pltpu.async_copy(rows[b], acc.at[idx_d.at[j]], ssem[b],
                                    add=True)

        gather(0, 0)
        gather(1, 1)

        def ring(g, carry):
            for i in range(4):
                j = 4 * g + i
                pltpu.make_async_copy(tab.at[idx_s.at[j]], rows[i],
                                      gsem[i]).wait()
                scatter(j, i)
                bi = (i + 2) % 4

                @pl.when(j >= 2)
                def _():
                    pltpu.make_async_copy(rows[bi], acc.at[idx_d.at[j]],
                                          ssem[bi]).wait()

                @pl.when(j + 2 < CHUNKS_PW)
                def _():
                    gather(j + 2, bi)
            return carry

        lax.fori_loop(0, CHUNKS_PW // 4, ring, 0)
        pltpu.make_async_copy(rows[2], acc.at[idx_d.at[0]], ssem[2]).wait()
        pltpu.make_async_copy(rows[3], acc.at[idx_d.at[0]], ssem[3]).wait()
        plsc.subcore_barrier()

        pltpu.sync_copy(acc.at[sl], out_hbm.at[c, sl])

    return segsum


_segsum1 = _make_segsum(fuse_combine=False)
_segsum2 = _make_segsum(fuse_combine=True)


def kernel(x, edge_index, W1, b1, W2, b2):
    src = edge_index[0].astype(jnp.int32)
    dst = edge_index[1].astype(jnp.int32)
    pad = EPAD - E
    srcp = jnp.concatenate([src, jnp.zeros((pad,), jnp.int32)])
    dstp = jnp.concatenate([dst, jnp.full((pad,), DUMMY, jnp.int32)])
    src3 = srcp.reshape(NW, CHUNKS_PW, CHUNK)
    dst3 = dstp.reshape(NW, CHUNKS_PW, CHUNK)
    b2r = b2.reshape(1, -1)

    y = pl.pallas_call(
        _mm1_body,
        grid=(NPAD // MM_BLK,),
        in_specs=[
            pl.BlockSpec((MM_BLK, F_IN), lambda i: (i, 0)),
            pl.BlockSpec((F_IN, F_HID), lambda i: (0, 0)),
        ],
        out_specs=pl.BlockSpec((MM_BLK, F_HID), lambda i: (i, 0)),
        out_shape=jax.ShapeDtypeStruct((NPAD, F_HID), jnp.float32),
    )(x, W1)

    p = _segsum1(y, src3, dst3)

    q = _segsum2(p, b1, src3, dst3)

    out = pl.pallas_call(
        _final_body,
        out_shape=jax.ShapeDtypeStruct((NPAD, b2.shape[0]), jnp.float32),
    )(q, W2, b2r)
    return out[:N]


# speedup vs baseline: 36.4334x; 1.1513x over previous
import functools

import jax
import jax.numpy as jnp
from jax import lax
from jax.experimental import pallas as pl
from jax.experimental.pallas import tpu as pltpu
from jax.experimental.pallas import tpu_sc as plsc

N = 10000
E = 160000
F_IN = 768
F_HID = 16

NC = 2
NS = 16
NW = NC * NS
CHUNK = 128
CHUNKS_PW = 40
E_PW = CHUNK * CHUNKS_PW
EPAD = NW * E_PW
NPAD = 10240
ROWS_PT = NPAD // NS
DUMMY = 10200

MM_BLK = 640


def _mm1_body(x_ref, w_ref, o_ref):
    o_ref[...] = jnp.dot(x_ref[...], w_ref[...],
                         preferred_element_type=jnp.float32)


def _final_body(q_ref, w_ref, b_ref, o_ref):
    agg = q_ref[0] + q_ref[1]
    outT = lax.dot_general(w_ref[...], agg, (((0,), (1,)), ((), ())),
                           preferred_element_type=jnp.float32)
    o_ref[...] = outT[:, :N] + b_ref[...]


def _make_segsum(fuse_combine):
    mesh = plsc.VectorSubcoreMesh(core_axis_name="c", subcore_axis_name="s")

    scratch = [
        pltpu.VMEM((E_PW,), jnp.int32),
        pltpu.VMEM((E_PW,), jnp.int32),
        [pltpu.VMEM((CHUNK, F_HID), jnp.float32)] * 4,
        pltpu.VMEM((ROWS_PT, F_HID), jnp.float32),
        pltpu.VMEM_SHARED((NPAD, F_HID), jnp.float32),
        pltpu.VMEM_SHARED((NPAD, F_HID), jnp.float32),
        [pltpu.SemaphoreType.DMA] * 4,
        [pltpu.SemaphoreType.DMA] * 4,
    ]
    if fuse_combine:
        scratch.insert(4, pltpu.VMEM((ROWS_PT, F_HID), jnp.float32))
        scratch.insert(5, pltpu.VMEM((ROWS_PT, F_HID), jnp.float32))
        scratch.insert(6, pltpu.VMEM((F_HID,), jnp.float32))

    @functools.partial(
        pl.kernel,
        mesh=mesh,
        out_type=jax.ShapeDtypeStruct((NC, NPAD, F_HID), jnp.float32),
        compiler_params=pltpu.CompilerParams(use_tc_tiling_on_sc=False),
        scratch_types=scratch,
    )
    def segsum(*refs):
        if fuse_combine:
            (p_hbm, b_hbm, edge_hbm, out_hbm,
             idx_s, idx_d, rows, zbuf, pb0, pb1, bbuf, acc, tab,
             gsem, ssem) = refs
        else:
            (table_hbm, edge_hbm, out_hbm,
             idx_s, idx_d, rows, zbuf, acc, tab,
             gsem, ssem) = refs
        c = lax.axis_index("c")
        s = lax.axis_index("s")
        w = c * NS + s
        sl = pl.ds(s * ROWS_PT, ROWS_PT)
        esl = pl.ds(w * E_PW, E_PW)

        cp_is = pltpu.async_copy(edge_hbm.at[0, esl], idx_s, gsem[0])
        cp_id = pltpu.async_copy(edge_hbm.at[1, esl], idx_d, gsem[1])
        if fuse_combine:
            cp_p0 = pltpu.async_copy(p_hbm.at[0, sl], pb0, gsem[2])
            cp_p1 = pltpu.async_copy(p_hbm.at[1, sl], pb1, gsem[3])
            cp_b = pltpu.async_copy(b_hbm, bbuf, ssem[0])
        else:
            cp_tab = pltpu.async_copy(table_hbm.at[sl], tab.at[sl], gsem[2])

        zero = jnp.zeros((F_HID,), jnp.float32)

        def zfill(i, carry):
            zbuf[i, :] = zero
            return carry

        lax.fori_loop(0, ROWS_PT, zfill, 0)

        if fuse_combine:
            cp_p0.wait()
            cp_p1.wait()
            cp_b.wait()
            bias = bbuf[...]

            def combine(i, carry):
                pb0[i, :] = jnp.maximum(pb0[i, :] + pb1[i, :] + bias, 0.0)
                return carry

            lax.fori_loop(0, ROWS_PT, combine, 0)
            pltpu.sync_copy(pb0, tab.at[sl])
        else:
            cp_tab.wait()
        cp_is.wait()
        cp_id.wait()
        pltpu.sync_copy(zbuf, acc.at[sl])
        plsc.subcore_barrier()

        def ich(ref, j):
            return ref.at[pl.ds(j * CHUNK, CHUNK)]

        def gather(j, b):
            return pltpu.async_copy(tab.at[ich(idx_s, j)], rows[b], gsem[b])

        def scatter(j, b):
            return pltpu.async_copy(rows[b], acc.at[ich(idx_d, j)], ssem[b],
                                    add=True)

        gather(0, 0)
        gather(1, 1)

        def ring(g, carry):
            for i in range(4):
                j = 4 * g + i
                pltpu.make_async_copy(tab.at[ich(idx_s, j)], rows[i],
                                      gsem[i]).wait()
                scatter(j, i)
                bi = (i + 2) % 4

                @pl.when(j >= 2)
                def _():
                    pltpu.make_async_copy(rows[bi], acc.at[ich(idx_d, j)],
                                          ssem[bi]).wait()

                @pl.when(j + 2 < CHUNKS_PW)
                def _():
                    gather(j + 2, bi)
            return carry

        lax.fori_loop(0, CHUNKS_PW // 4, ring, 0)
        pltpu.make_async_copy(rows[2], acc.at[ich(idx_d, 0)], ssem[2]).wait()
        pltpu.make_async_copy(rows[3], acc.at[ich(idx_d, 0)], ssem[3]).wait()
        plsc.subcore_barrier()

        pltpu.sync_copy(acc.at[sl], out_hbm.at[c, sl])

    return segsum


_segsum1 = _make_segsum(fuse_combine=False)
_segsum2 = _make_segsum(fuse_combine=True)


def kernel(x, edge_index, W1, b1, W2, b2):
    edges = jnp.pad(edge_index.astype(jnp.int32), ((0, 0), (0, EPAD - E)),
                    constant_values=DUMMY)
    b2r = b2.reshape(-1, 1)

    y = pl.pallas_call(
        _mm1_body,
        grid=(NPAD // MM_BLK,),
        in_specs=[
            pl.BlockSpec((MM_BLK, F_IN), lambda i: (i, 0)),
            pl.BlockSpec((F_IN, F_HID), lambda i: (0, 0)),
        ],
        out_specs=pl.BlockSpec((MM_BLK, F_HID), lambda i: (i, 0)),
        out_shape=jax.ShapeDtypeStruct((NPAD, F_HID), jnp.float32),
    )(x, W1)

    p = _segsum1(y, edges)

    q = _segsum2(p, b1, edges)

    outT = pl.pallas_call(
        _final_body,
        out_shape=jax.ShapeDtypeStruct((b2.shape[0], N), jnp.float32),
    )(q, W2, b2r)
    return outT.T
